# Initial kernel scaffold; baseline (speedup 1.0000x reference)
import jax
import jax.numpy as jnp
from jax.experimental import pallas as pl


def kernel(x, edge_index, W1, b1, g1, be1, W2, b2, W3, b3, g2, be2, W4, b4, W5, b5, g3, be3, W6, b6, Wf1, bf1, Wf2, bf2):
    raise NotImplementedError("write your pallas kernel here")


# speedup vs baseline: 12.2913x; 12.2913x over previous
import jax
import jax.numpy as jnp
from jax import lax
from jax.experimental import pallas as pl
from jax.experimental.pallas import tpu as pltpu
from jax.experimental.pallas import tpu_sc as plsc

_N = 10000
_E = 320000
_EPS = 1e-5

_NC, _NS = 2, 16
_NW = _NC * _NS
_EW = _E // _NW
_CH = 80
_NCH = _EW // _CH
_NP = 10112
_RS = _NP // _NS


def _make_seg_sum(width):
    def body(h, srcs, dsts, zeros, out, src_v, dst_v, rows0, rows1, sem0,
             sem1, acc):
        c = lax.axis_index("c")
        s = lax.axis_index("s")
        wid = c * _NS + s
        pltpu.sync_copy(zeros.at[pl.ds(s * _RS, _RS)],
                        acc.at[pl.ds(s * _RS, _RS)])
        pltpu.sync_copy(srcs.at[wid], src_v)
        pltpu.sync_copy(dsts.at[wid], dst_v)
        plsc.subcore_barrier()

        pltpu.async_copy(h.at[src_v.at[0]], rows0, sem0)

        def step(i, carry):
            j0 = 2 * i
            pltpu.async_copy(h.at[src_v.at[j0 + 1]], rows1, sem1)
            pltpu.make_async_copy(h.at[src_v.at[j0]], rows0, sem0).wait()
            pltpu.sync_copy(rows0, acc.at[dst_v.at[j0]], add=True)
            pltpu.async_copy(h.at[src_v.at[j0 + 2]], rows0, sem0)
            pltpu.make_async_copy(h.at[src_v.at[j0 + 1]], rows1, sem1).wait()
            pltpu.sync_copy(rows1, acc.at[dst_v.at[j0 + 1]], add=True)
            return carry

        lax.fori_loop(0, (_NCH - 1) // 2, step, 0)
        pltpu.make_async_copy(h.at[src_v.at[_NCH - 1]], rows0, sem0).wait()
        pltpu.sync_copy(rows0, acc.at[dst_v.at[_NCH - 1]], add=True)

        plsc.subcore_barrier()
        pltpu.sync_copy(acc.at[pl.ds(s * _RS, _RS)],
                        out.at[c, pl.ds(s * _RS, _RS)])

    return pl.kernel(
        body,
        out_type=jax.ShapeDtypeStruct((_NC, _NP, width), jnp.float32),
        mesh=plsc.VectorSubcoreMesh(core_axis_name="c", subcore_axis_name="s",
                                    num_cores=_NC, num_subcores=_NS),
        scratch_types=[
            pltpu.VMEM((_NCH, _CH), jnp.int32),
            pltpu.VMEM((_NCH, _CH), jnp.int32),
            pltpu.VMEM((_CH, width), jnp.float32),
            pltpu.VMEM((_CH, width), jnp.float32),
            pltpu.SemaphoreType.DMA,
            pltpu.SemaphoreType.DMA,
            pltpu.VMEM_SHARED((_NP, width), jnp.float32),
        ],
        compiler_params=pltpu.CompilerParams(use_tc_tiling_on_sc=False),
    )


_seg128 = _make_seg_sum(128)
_seg32 = _make_seg_sum(32)


def _bn_relu(z, g, be):
    mu = jnp.mean(z, axis=0, keepdims=True)
    var = jnp.var(z, axis=0, keepdims=True)
    return jnp.maximum(g * (z - mu) / jnp.sqrt(var + _EPS) + be, 0.0)


def _gin_body(h, p, wa, ba, g, be, wb, bb, xout, sout):
    z = jnp.dot(h[...] + p[0, :_N] + p[1, :_N], wa[...]) + ba[...]
    h1 = _bn_relu(z, g[...], be[...])
    x = jnp.maximum(jnp.dot(h1, wb[...]) + bb[...], 0.0)
    xout[...] = x
    sout[...] = jnp.sum(x, axis=1, keepdims=True)


def _gin_call(h, p, Wa, ba, g, be, Wb, bb):
    fn = pl.pallas_call(
        _gin_body,
        out_shape=(jax.ShapeDtypeStruct((_N, Wb.shape[1]), jnp.float32),
                   jax.ShapeDtypeStruct((_N, 1), jnp.float32)))
    r2 = lambda v: v.reshape(1, -1)
    return fn(h, p, Wa, r2(ba), r2(g), r2(be), Wb, r2(bb))


def _final_body(h, p, w5, b5, g, be, w6, b6, x1s, x2s, wf1, bf1, wf2, bf2,
                r_ref, x3s_ref):
    z = jnp.dot(h[...] + p[0, :_N] + p[1, :_N], w5[...]) + b5[...]
    h1 = _bn_relu(z, g[...], be[...])
    x3 = jnp.maximum(jnp.dot(h1, w6[...]) + b6[...], 0.0)
    x3s_ref[...] = x3
    cgate = 1.0 / (1.0 + jnp.exp(-x3))
    wf1v = wf1[...]
    t = (x1s[...] * wf1v[0:1, :] + x2s[...] * wf1v[1:2, :]
         + x3 * wf1v[2:3, :] + cgate * wf1v[3:4, :] + bf1[...])
    t = jnp.maximum(t, 0.0)
    r_ref[...] = jnp.dot(t, wf2[...]) + bf2[...]


_final = pl.pallas_call(
    _final_body,
    out_shape=(jax.ShapeDtypeStruct((_N, 1), jnp.float32),
               jax.ShapeDtypeStruct((_N, 1), jnp.float32)))


def kernel(x, edge_index, W1, b1, g1, be1, W2, b2, W3, b3, g2, be2, W4, b4,
           W5, b5, g3, be3, W6, b6, Wf1, bf1, Wf2, bf2):
    ei = edge_index.astype(jnp.int32)
    srcs = ei[0].reshape(_NW, _NCH, _CH)
    dsts = ei[1].reshape(_NW, _NCH, _CH)
    z128 = jnp.zeros((_NP, 128), jnp.float32)
    z32 = jnp.zeros((_NP, 32), jnp.float32)
    r2 = lambda v: v.reshape(1, -1)

    p1 = _seg128(x, srcs, dsts, z128)
    x1, x1s = _gin_call(x, p1, W1, b1, g1, be1, W2, b2)
    p2 = _seg32(x1, srcs, dsts, z32)
    x2, x2s = _gin_call(x1, p2, W3, b3, g2, be2, W4, b4)
    p3 = _seg32(x2, srcs, dsts, z32)
    r, x3s = _final(x2, p3, W5, r2(b5), r2(g3), r2(be3), W6, r2(b6),
                    x1s, x2s, Wf1, r2(bf1), Wf2, r2(bf2))
    return (r, x3s)
